# baseline (speedup 1.0000x reference)
import functools

import jax
import jax.numpy as jnp
from jax import lax
from jax.experimental import pallas as pl
from jax.experimental.pallas import tpu as pltpu
from jax.experimental.pallas import tpu_sc as plsc

CH = 128
P = 256
GRP = 8
C_IN = 128
KNUM = 16
K_NB = 16
ROW = 128
FL = C_IN // 2
BNP = 25600


def _sc_gather(table, idx, n_edges):
    info = plsc.get_sparse_core_info()
    nw = info.num_cores * info.num_subcores
    n_chunks = n_edges // CH
    chunks_per_w = n_chunks // nw
    row = table.shape[1]
    mesh = plsc.VectorSubcoreMesh(core_axis_name="c", subcore_axis_name="s")

    @functools.partial(
        pl.kernel,
        mesh=mesh,
        out_type=jax.ShapeDtypeStruct((n_edges, row), jnp.float32),
        scratch_types=[
            pltpu.VMEM((2, CH), jnp.int32),
            pltpu.VMEM((2, CH, row), jnp.float32),
            pltpu.SemaphoreType.DMA,
            pltpu.SemaphoreType.DMA,
            pltpu.SemaphoreType.DMA,
        ],
    )
    def gather_kernel(table_hbm, idx_hbm, out_hbm, idx_v, rows_v,
                      sem_i, sem_g, sem_o):
        wid = lax.axis_index("s") * info.num_cores + lax.axis_index("c")

        def idx_start(i, buf):
            off = (wid + i * nw) * CH
            pltpu.async_copy(idx_hbm.at[pl.ds(off, CH)], idx_v.at[buf], sem_i)

        def out_start(i, buf):
            off = (wid + i * nw) * CH
            pltpu.async_copy(rows_v.at[buf], out_hbm.at[pl.ds(off, CH)], sem_o)

        def drain(src, dst, sem):
            pltpu.make_async_copy(src, dst, sem).wait()

        idx_start(0, 0)

        def body(i, carry):
            buf = lax.rem(i, 2)
            drain(idx_hbm.at[pl.ds(0, CH)], idx_v.at[buf], sem_i)

            @pl.when(i + 1 < chunks_per_w)
            def _():
                idx_start(i + 1, 1 - buf)

            @pl.when(i >= 2)
            def _():
                drain(rows_v.at[buf], out_hbm.at[pl.ds(0, CH)], sem_o)

            pltpu.async_copy(table_hbm.at[idx_v.at[buf]], rows_v.at[buf],
                             sem_g).wait()
            out_start(i, buf)
            return carry

        lax.fori_loop(0, chunks_per_w, body, 0)
        drain(rows_v.at[0], out_hbm.at[pl.ds(0, CH)], sem_o)
        drain(rows_v.at[1], out_hbm.at[pl.ds(0, CH)], sem_o)

    return gather_kernel(table, idx)


def _dot(a, b):
    return lax.dot_general(a, b, (((1,), (0,)), ((), ())),
                           preferred_element_type=jnp.float32)


def _tc_body(g_ref, p_ref, opts_ref, w1_ref, b1_ref, w2_ref, b2_ref, w3_ref,
             b3_ref, wf_ref, bias_ref, out_ref, d_ref, z_ref, s_ref):
    ek = P * K_NB
    opts_rep = jnp.broadcast_to(opts_ref[...][:, None, :],
                                (P, K_NB, 3)).reshape(ek, 3)
    rel = p_ref[...] - opts_rep
    h = jnp.maximum(_dot(rel, w1_ref[...]) + b1_ref[...], 0.0)
    h = jnp.maximum(_dot(h, w2_ref[...]) + b2_ref[...], 0.0)
    d_ref[...] = jnp.maximum(_dot(h, w3_ref[...]) + b3_ref[...], 0.0)
    @pl.when(pl.program_id(0) == 0)
    def _():
        s_ref[...] = jnp.zeros((GRP * K_NB, GRP * KNUM), jnp.bfloat16)

    for g in range(P // GRP):
        rows = g * GRP * K_NB
        for q in range(GRP):
            s_ref[q * K_NB:(q + 1) * K_NB, q * KNUM:(q + 1) * KNUM] = (
                d_ref[rows + q * K_NB:rows + (q + 1) * K_NB, :]
                .astype(jnp.bfloat16))
        fg = g_ref[rows:rows + GRP * K_NB, 0:C_IN]
        zg = lax.dot_general(s_ref[...], fg, (((0,), (0,)), ((), ())),
                             preferred_element_type=jnp.float32)
        z_ref[g * GRP:(g + 1) * GRP, :, :] = zg.reshape(GRP, KNUM, C_IN)
    out = bias_ref[...]
    for j in range(KNUM):
        out += _dot(z_ref[:, j, :], wf_ref[j])
    out_ref[...] = out


def _tc_compute(g, pg, opts, w1pp, b1p, w2, b2, w3, b3, wf, bias):
    n_blocks = BNP // P
    full = lambda *shape: pl.BlockSpec(shape, lambda i: (0,) * len(shape))
    return pl.pallas_call(
        _tc_body,
        grid=(n_blocks,),
        in_specs=[
            pl.BlockSpec((P * K_NB, 2 * ROW), lambda i: (i, 0)),
            pl.BlockSpec((P * K_NB, 3), lambda i: (i, 0)),
            pl.BlockSpec((P, 3), lambda i: (i, 0)),
            full(3, 32), full(1, 32), full(32, 16), full(1, 16),
            full(16, 16), full(1, 16), full(KNUM, C_IN, 128), full(1, 128),
        ],
        out_specs=pl.BlockSpec((P, 128), lambda i: (i, 0)),
        out_shape=jax.ShapeDtypeStruct((BNP, 128), jnp.float32),
        scratch_shapes=[pltpu.VMEM((P * K_NB, KNUM), jnp.float32),
                        pltpu.VMEM((P, KNUM, C_IN), jnp.float32),
                        pltpu.VMEM((GRP * K_NB, GRP * KNUM), jnp.bfloat16)],
    )(g, pg, opts, w1pp, b1p, w2, b2, w3, b3, wf, bias)


def kernel(features, input_pts, neighbor_num, output_pts, indices_, weight,
           bias, centers, W1, b1, W2, b2, W3, b3):
    b_sz, n_pts, c_in = features.shape
    k_nb = indices_.shape[2]
    bn = b_sz * n_pts
    dim = input_pts.shape[2]

    fpacked = lax.bitcast_convert_type(
        features.reshape(bn, c_in).astype(jnp.bfloat16).reshape(bn, FL, 2),
        jnp.float32)
    table = jnp.concatenate(
        [fpacked,
         jnp.pad(input_pts.reshape(bn, dim), ((0, 0), (0, ROW - FL - dim)))],
        axis=1)
    add = (jnp.arange(b_sz, dtype=indices_.dtype) * n_pts)[:, None, None]
    idx_f = (indices_ + add).reshape(-1).astype(jnp.int32)
    idx_f = jnp.pad(idx_f, (0, (BNP - bn) * k_nb))

    gathered = _sc_gather(table, idx_f, BNP * k_nb)
    gb = lax.bitcast_convert_type(gathered, jnp.bfloat16)
    gb = gb.reshape(BNP * k_nb, 2 * ROW)
    pts_g = gathered[:, FL:FL + dim]

    w1pp = W1.reshape(dim, KNUM, W1.shape[1]).sum(axis=1)
    b1p = b1 - centers.reshape(-1) @ W1
    wf = jnp.transpose(weight, (1, 0, 2)) / float(k_nb)
    opts = jnp.pad(output_pts.reshape(bn, dim), ((0, BNP - bn), (0, 0)))

    out = _tc_compute(gb, pts_g, opts, w1pp, b1p[None],
                      W2, b2[None], W3, b3[None], wf, bias[None])
    return out[:bn].reshape(b_sz, n_pts, weight.shape[2]), output_pts


# speedup vs baseline: 2.2671x; 2.2671x over previous
import functools

import jax
import jax.numpy as jnp
from jax import lax
from jax.experimental import pallas as pl
from jax.experimental.pallas import tpu as pltpu
from jax.experimental.pallas import tpu_sc as plsc

CH = 128
P = 256
GRP = 8
C_IN = 128
KNUM = 16
K_NB = 16
ROW = 128
FL = C_IN // 2
BNP = 25600


def _sc_gather(table, idx, n_edges):
    info = plsc.get_sparse_core_info()
    nw = info.num_cores * info.num_subcores
    n_chunks = n_edges // CH
    chunks_per_w = n_chunks // nw
    row = table.shape[1]
    mesh = plsc.VectorSubcoreMesh(core_axis_name="c", subcore_axis_name="s")

    @functools.partial(
        pl.kernel,
        mesh=mesh,
        out_type=jax.ShapeDtypeStruct((n_edges, row), jnp.float32),
        scratch_types=[
            pltpu.VMEM((2, CH), jnp.int32),
            pltpu.VMEM((2, CH, row), jnp.float32),
            pltpu.SemaphoreType.DMA,
            pltpu.SemaphoreType.DMA,
            pltpu.SemaphoreType.DMA,
        ],
    )
    def gather_kernel(table_hbm, idx_hbm, out_hbm, idx_v, rows_v,
                      sem_i, sem_g, sem_o):
        wid = lax.axis_index("s") * info.num_cores + lax.axis_index("c")

        def idx_start(i, buf):
            off = (wid + i * nw) * CH
            pltpu.async_copy(idx_hbm.at[pl.ds(off, CH)], idx_v.at[buf], sem_i)

        def out_start(i, buf):
            off = (wid + i * nw) * CH
            pltpu.async_copy(rows_v.at[buf], out_hbm.at[pl.ds(off, CH)], sem_o)

        def drain(src, dst, sem):
            pltpu.make_async_copy(src, dst, sem).wait()

        idx_start(0, 0)

        def body(i, carry):
            buf = lax.rem(i, 2)
            drain(idx_hbm.at[pl.ds(0, CH)], idx_v.at[buf], sem_i)

            @pl.when(i + 1 < chunks_per_w)
            def _():
                idx_start(i + 1, 1 - buf)

            @pl.when(i >= 2)
            def _():
                drain(rows_v.at[buf], out_hbm.at[pl.ds(0, CH)], sem_o)

            pltpu.async_copy(table_hbm.at[idx_v.at[buf]], rows_v.at[buf],
                             sem_g).wait()
            out_start(i, buf)
            return carry

        lax.fori_loop(0, chunks_per_w, body, 0)
        drain(rows_v.at[0], out_hbm.at[pl.ds(0, CH)], sem_o)
        drain(rows_v.at[1], out_hbm.at[pl.ds(0, CH)], sem_o)

    return gather_kernel(table, idx)


def _dot(a, b):
    return lax.dot_general(a, b, (((1,), (0,)), ((), ())),
                           preferred_element_type=jnp.float32)


def _tc_body(g_ref, opts_ref, w1_ref, b1_ref, w2_ref, b2_ref, w3_ref,
             b3_ref, wf_ref, bias_ref, out_ref, d_ref, z_ref, s_ref, f_ref):
    ek = P * K_NB
    opts_rep = jnp.broadcast_to(opts_ref[...][:, None, :],
                                (P, K_NB, 3)).reshape(ek, 3)
    rel = g_ref[:, FL:FL + 3] - opts_rep
    h = jnp.maximum(_dot(rel, w1_ref[...]) + b1_ref[...], 0.0)
    h = jnp.maximum(_dot(h, w2_ref[...]) + b2_ref[...], 0.0)
    d_ref[...] = jnp.maximum(_dot(h, w3_ref[...]) + b3_ref[...], 0.0)
    @pl.when(pl.program_id(0) == 0)
    def _():
        s_ref[...] = jnp.zeros((GRP * K_NB, GRP * KNUM), jnp.bfloat16)

    g_i = pltpu.bitcast(g_ref[:, 0:FL], jnp.int32)
    for half in range(2):
        fh_i = pltpu.unpack_elementwise(
            g_i, index=half, packed_dtype=jnp.int16, unpacked_dtype=jnp.int32)
        f_ref[:, half * FL:(half + 1) * FL] = pltpu.bitcast(
            fh_i << 16, jnp.float32).astype(jnp.bfloat16)

    for g in range(P // GRP):
        rows = g * GRP * K_NB
        for q in range(GRP):
            s_ref[q * K_NB:(q + 1) * K_NB, q * KNUM:(q + 1) * KNUM] = (
                d_ref[rows + q * K_NB:rows + (q + 1) * K_NB, :]
                .astype(jnp.bfloat16))
        fg = f_ref[rows:rows + GRP * K_NB, :]
        zg = lax.dot_general(s_ref[...], fg, (((0,), (0,)), ((), ())),
                             preferred_element_type=jnp.float32)
        z_ref[g * GRP:(g + 1) * GRP, :, :] = zg.reshape(GRP, KNUM, C_IN)
    out = bias_ref[...]
    for j in range(KNUM):
        out += _dot(z_ref[:, j, :], wf_ref[j])
    out_ref[...] = out


def _tc_compute(g, opts, w1pp, b1p, w2, b2, w3, b3, wf, bias):
    n_blocks = BNP // P
    full = lambda *shape: pl.BlockSpec(shape, lambda i: (0,) * len(shape))
    return pl.pallas_call(
        _tc_body,
        grid=(n_blocks,),
        in_specs=[
            pl.BlockSpec((P * K_NB, ROW), lambda i: (i, 0)),
            pl.BlockSpec((P, 3), lambda i: (i, 0)),
            full(3, 32), full(1, 32), full(32, 16), full(1, 16),
            full(16, 16), full(1, 16), full(KNUM, C_IN, 128), full(1, 128),
        ],
        out_specs=pl.BlockSpec((P, 128), lambda i: (i, 0)),
        out_shape=jax.ShapeDtypeStruct((BNP, 128), jnp.float32),
        scratch_shapes=[pltpu.VMEM((P * K_NB, KNUM), jnp.float32),
                        pltpu.VMEM((P, KNUM, C_IN), jnp.float32),
                        pltpu.VMEM((GRP * K_NB, GRP * KNUM), jnp.bfloat16),
                        pltpu.VMEM((P * K_NB, C_IN), jnp.bfloat16)],
    )(g, opts, w1pp, b1p, w2, b2, w3, b3, wf, bias)


def kernel(features, input_pts, neighbor_num, output_pts, indices_, weight,
           bias, centers, W1, b1, W2, b2, W3, b3):
    b_sz, n_pts, c_in = features.shape
    k_nb = indices_.shape[2]
    bn = b_sz * n_pts
    dim = input_pts.shape[2]

    fpacked = lax.bitcast_convert_type(
        features.reshape(bn, c_in).astype(jnp.bfloat16).reshape(bn, FL, 2),
        jnp.float32)
    table = jnp.concatenate(
        [fpacked,
         jnp.pad(input_pts.reshape(bn, dim), ((0, 0), (0, ROW - FL - dim)))],
        axis=1)
    add = (jnp.arange(b_sz, dtype=indices_.dtype) * n_pts)[:, None, None]
    idx_f = (indices_ + add).reshape(-1).astype(jnp.int32)
    idx_f = jnp.pad(idx_f, (0, (BNP - bn) * k_nb))

    gathered = _sc_gather(table, idx_f, BNP * k_nb)

    w1pp = W1.reshape(dim, KNUM, W1.shape[1]).sum(axis=1)
    b1p = b1 - centers.reshape(-1) @ W1
    perm = jnp.concatenate([jnp.arange(0, c_in, 2), jnp.arange(1, c_in, 2)])
    wf = (jnp.transpose(weight, (1, 0, 2)) / float(k_nb))[:, perm, :]
    opts = jnp.pad(output_pts.reshape(bn, dim), ((0, BNP - bn), (0, 0)))

    out = _tc_compute(gathered, opts, w1pp, b1p[None],
                      W2, b2[None], W3, b3[None], wf, bias[None])
    return out[:bn].reshape(b_sz, n_pts, weight.shape[2]), output_pts


# speedup vs baseline: 3.1131x; 1.3732x over previous
import functools

import jax
import jax.numpy as jnp
from jax import lax
from jax.experimental import pallas as pl
from jax.experimental.pallas import tpu as pltpu
from jax.experimental.pallas import tpu_sc as plsc

CH = 128
P = 200
GRP = 8
C_IN = 128
KNUM = 16
K_NB = 16
ROW = 128
FL = C_IN // 2
BNP = 25000


def _sc_gather(table, idx, n_edges):
    info = plsc.get_sparse_core_info()
    nw = info.num_cores * info.num_subcores
    n_chunks = n_edges // CH
    chunks_per_w = (n_chunks + nw - 1) // nw
    row = table.shape[1]
    mesh = plsc.VectorSubcoreMesh(core_axis_name="c", subcore_axis_name="s")

    @functools.partial(
        pl.kernel,
        mesh=mesh,
        out_type=jax.ShapeDtypeStruct((n_edges, row), jnp.float32),
        scratch_types=[
            pltpu.VMEM((2, CH), jnp.int32),
            pltpu.VMEM((2, CH, row), jnp.float32),
            pltpu.SemaphoreType.DMA,
            pltpu.SemaphoreType.DMA,
            pltpu.SemaphoreType.DMA,
        ],
    )
    def gather_kernel(table_hbm, idx_hbm, out_hbm, idx_v, rows_v,
                      sem_i, sem_g, sem_o):
        wid = lax.axis_index("s") * info.num_cores + lax.axis_index("c")

        def idx_start(i, buf):
            off = (wid + i * nw) * CH
            pltpu.async_copy(idx_hbm.at[pl.ds(off, CH)], idx_v.at[buf], sem_i)

        def out_start(i, buf):
            off = (wid + i * nw) * CH
            pltpu.async_copy(rows_v.at[buf], out_hbm.at[pl.ds(off, CH)], sem_o)

        def drain(src, dst, sem):
            pltpu.make_async_copy(src, dst, sem).wait()

        idx_start(0, 0)

        def body(i, carry):
            buf = lax.rem(i, 2)
            @pl.when(i >= 2)
            def _():
                drain(rows_v.at[buf], out_hbm.at[pl.ds(0, CH)], sem_o)

            @pl.when(wid + i * nw < n_chunks)
            def _():
                drain(idx_hbm.at[pl.ds(0, CH)], idx_v.at[buf], sem_i)

                @pl.when(wid + (i + 1) * nw < n_chunks)
                def _():
                    idx_start(i + 1, 1 - buf)

                pltpu.async_copy(table_hbm.at[idx_v.at[buf]], rows_v.at[buf],
                                 sem_g).wait()
                out_start(i, buf)
            return carry

        lax.fori_loop(0, chunks_per_w, body, 0)

        for tail in (chunks_per_w - 2, chunks_per_w - 1):
            @pl.when(wid + tail * nw < n_chunks)
            def _():
                drain(rows_v.at[0], out_hbm.at[pl.ds(0, CH)], sem_o)

    return gather_kernel(table, idx)


def _dot(a, b):
    return lax.dot_general(a, b, (((1,), (0,)), ((), ())),
                           preferred_element_type=jnp.float32)


def _tc_body(g_ref, opts_ref, w1_ref, b1_ref, w2_ref, b2_ref, w3_ref,
             b3_ref, wf_ref, bias_ref, out_ref, d_ref, z_ref, s_ref, f_ref):
    ek = P * K_NB
    opts_rep = jnp.broadcast_to(opts_ref[...][:, None, :],
                                (P, K_NB, 3)).reshape(ek, 3)
    rel = g_ref[:, FL:FL + 3] - opts_rep
    h = jnp.maximum(_dot(rel, w1_ref[...]) + b1_ref[...], 0.0)
    h = jnp.maximum(_dot(h, w2_ref[...]) + b2_ref[...], 0.0)
    d_ref[...] = jnp.maximum(_dot(h, w3_ref[...]) + b3_ref[...], 0.0)
    @pl.when(pl.program_id(0) == 0)
    def _():
        s_ref[...] = jnp.zeros((GRP * K_NB, GRP * KNUM), jnp.bfloat16)

    g_i = pltpu.bitcast(g_ref[:, 0:FL], jnp.int32)
    for half in range(2):
        fh_i = pltpu.unpack_elementwise(
            g_i, index=half, packed_dtype=jnp.int16, unpacked_dtype=jnp.int32)
        f_ref[:, half * FL:(half + 1) * FL] = pltpu.bitcast(
            fh_i << 16, jnp.float32).astype(jnp.bfloat16)

    for g in range(P // GRP):
        rows = g * GRP * K_NB
        for q in range(GRP):
            s_ref[q * K_NB:(q + 1) * K_NB, q * KNUM:(q + 1) * KNUM] = (
                d_ref[rows + q * K_NB:rows + (q + 1) * K_NB, :]
                .astype(jnp.bfloat16))
        fg = f_ref[rows:rows + GRP * K_NB, :]
        zg = lax.dot_general(s_ref[...], fg, (((0,), (0,)), ((), ())),
                             preferred_element_type=jnp.float32)
        z_ref[g * GRP:(g + 1) * GRP, :, :] = zg.reshape(GRP, KNUM, C_IN)
    out = bias_ref[...]
    for j in range(KNUM):
        out += _dot(z_ref[:, j, :], wf_ref[j])
    out_ref[...] = out


def _tc_compute(g, opts, w1pp, b1p, w2, b2, w3, b3, wf, bias):
    n_blocks = BNP // P
    full = lambda *shape: pl.BlockSpec(shape, lambda i: (0,) * len(shape))
    return pl.pallas_call(
        _tc_body,
        grid=(n_blocks,),
        in_specs=[
            pl.BlockSpec((P * K_NB, ROW), lambda i: (i, 0)),
            pl.BlockSpec((P, 3), lambda i: (i, 0)),
            full(3, 32), full(1, 32), full(32, 16), full(1, 16),
            full(16, 16), full(1, 16), full(KNUM, C_IN, 128), full(1, 128),
        ],
        out_specs=pl.BlockSpec((P, 128), lambda i: (i, 0)),
        out_shape=jax.ShapeDtypeStruct((BNP, 128), jnp.float32),
        scratch_shapes=[pltpu.VMEM((P * K_NB, KNUM), jnp.float32),
                        pltpu.VMEM((P, KNUM, C_IN), jnp.float32),
                        pltpu.VMEM((GRP * K_NB, GRP * KNUM), jnp.bfloat16),
                        pltpu.VMEM((P * K_NB, C_IN), jnp.bfloat16)],
    )(g, opts, w1pp, b1p, w2, b2, w3, b3, wf, bias)


def kernel(features, input_pts, neighbor_num, output_pts, indices_, weight,
           bias, centers, W1, b1, W2, b2, W3, b3):
    b_sz, n_pts, c_in = features.shape
    k_nb = indices_.shape[2]
    bn = b_sz * n_pts
    dim = input_pts.shape[2]

    fpacked = lax.bitcast_convert_type(
        features.reshape(bn, c_in).astype(jnp.bfloat16).reshape(bn, FL, 2),
        jnp.float32)
    table = jnp.concatenate(
        [fpacked,
         jnp.pad(input_pts.reshape(bn, dim), ((0, 0), (0, ROW - FL - dim)))],
        axis=1)
    add = (jnp.arange(b_sz, dtype=indices_.dtype) * n_pts)[:, None, None]
    idx_f = (indices_ + add).reshape(-1).astype(jnp.int32)

    gathered = _sc_gather(table, idx_f, BNP * k_nb)

    w1pp = W1.reshape(dim, KNUM, W1.shape[1]).sum(axis=1)
    b1p = b1 - centers.reshape(-1) @ W1
    perm = jnp.concatenate([jnp.arange(0, c_in, 2), jnp.arange(1, c_in, 2)])
    wf = (jnp.transpose(weight, (1, 0, 2)) / float(k_nb))[:, perm, :]
    opts = output_pts.reshape(bn, dim)

    out = _tc_compute(gathered, opts, w1pp, b1p[None],
                      W2, b2[None], W3, b3[None], wf, bias[None])
    return out.reshape(b_sz, n_pts, weight.shape[2]), output_pts
